# baseline (speedup 1.0000x reference)
import functools

import jax
import jax.numpy as jnp
from jax import lax
from jax.experimental import pallas as pl
from jax.experimental.pallas import tpu as pltpu
from jax.experimental.pallas import tpu_sc as plsc

NC = 2
NS = 16
NW = NC * NS

VOCAB = 1000000
DIM = 32
GRP = 4

SLEN = 1024
NBUF = 3


def _tc_transpose():
    B4 = 2048
    NB = 123

    def body(x0, x1, x2, x3, o_ref):
        x = jnp.concatenate(
            [x0[...], x1[...], x2[...], x3[...]], axis=0)
        eye = jnp.eye(128, dtype=jnp.float32)
        o_ref[...] = jax.lax.dot_general(
            x, eye, (((0,), (0,)), ((), ())),
            preferred_element_type=jnp.float32,
            precision=lax.Precision.HIGHEST)

    def in_spec(m):
        return pl.BlockSpec(
            (32, B4), lambda g, m=m: (0, jnp.minimum(g + m * NB, 488)))

    return pl.pallas_call(
        body,
        grid=(NB,),
        in_specs=[in_spec(m) for m in range(4)],
        out_specs=pl.BlockSpec((B4, 128), lambda g: (g, 0)),
        out_shape=jax.ShapeDtypeStruct((NB * B4, 128), jnp.float32),
        compiler_params=pltpu.CompilerParams(
            dimension_semantics=("arbitrary",)),
    )


def _tc_outpack():
    def body(g_ref, o_ref):
        eye = jnp.eye(128, dtype=jnp.float32)
        t = jax.lax.dot_general(
            g_ref[...], eye, (((0,), (0,)), ((), ())),
            preferred_element_type=jnp.float32,
            precision=lax.Precision.HIGHEST)
        o_ref[...] = t.reshape(1, 4, 1, 8, 128)

    return pl.pallas_call(
        body,
        grid=(20, 128),
        in_specs=[pl.BlockSpec((128, 32), lambda l, bh: (l * 128 + bh, 0))],
        out_specs=pl.BlockSpec(
            (1, 4, 1, 8, 128), lambda l, bh: (l, 0, bh, 0, 0)),
        out_shape=jax.ShapeDtypeStruct((20, 4, 128, 8, 128), jnp.float32),
        compiler_params=pltpu.CompilerParams(
            dimension_semantics=("arbitrary", "arbitrary")),
    )


def _make_gather(n_rows, dim, slots):
    mesh = plsc.VectorSubcoreMesh(core_axis_name="c", subcore_axis_name="s")

    @functools.partial(
        pl.kernel,
        mesh=mesh,
        out_type=jax.ShapeDtypeStruct((NW, slots, SLEN, dim), jnp.float32),
        scratch_types=[
            pltpu.VMEM((slots, SLEN), jnp.int32),
            pltpu.VMEM((NBUF, SLEN, dim), jnp.float32),
            [pltpu.SemaphoreType.DMA] * NBUF,
            [pltpu.SemaphoreType.DMA] * NBUF,
        ],
        compiler_params=pltpu.CompilerParams(use_tc_tiling_on_sc=False),
    )
    def grab(table_hbm, ids_hbm, out_hbm, idx_v, rows_v, gsems, osems):
        wid = lax.axis_index("s") * NC + lax.axis_index("c")
        pltpu.sync_copy(ids_hbm.at[wid], idx_v)

        def fire_gather(g):
            return pltpu.async_copy(
                table_hbm.at[idx_v.at[g]], rows_v.at[g % NBUF], gsems[g % NBUF]
            )

        gh = {g: fire_gather(g) for g in range(min(2, slots))}
        oh = {}
        for g in range(slots):
            gh.pop(g).wait()
            oh[g] = pltpu.async_copy(
                rows_v.at[g % NBUF], out_hbm.at[wid, g], osems[g % NBUF]
            )
            if g + 2 < slots:
                if g - 1 >= 0:
                    oh.pop(g - 1).wait()
                gh[g + 2] = fire_gather(g + 2)
        for h in oh.values():
            h.wait()

    return grab


def kernel(input, table):
    ids = input
    n = ids.shape[0] * ids.shape[1]
    dim = table.shape[1]
    slots = n // (NW * SLEN)
    flat = ids.T.reshape(-1)
    flat = 4 * (flat % 251904) + flat // 251904
    ids3 = flat.reshape(NW, slots, SLEN)
    tt = table.T
    packed = _tc_transpose()(tt, tt, tt, tt)
    tlin = packed.reshape(4 * 251904, dim)
    rows = _make_gather(4 * 251904, dim, slots)(tlin, ids3)
    out6 = _tc_outpack()(rows.reshape(n, dim))
    out = jnp.transpose(out6, (2, 4, 0, 1, 3)).reshape(ids.shape + (dim,))
    return out


# speedup vs baseline: 3.9208x; 3.9208x over previous
import functools

import jax
import jax.numpy as jnp
from jax import lax
from jax.experimental import pallas as pl
from jax.experimental.pallas import tpu as pltpu
from jax.experimental.pallas import tpu_sc as plsc

NC = 2
NS = 16
NW = NC * NS

VOCAB = 1000000
DIM = 32
GRP = 4

SLEN = 1024
NBUF = 3


def _tc_transpose():
    B4 = 2048
    NB = 123

    def body(x0, x1, x2, x3, o_ref):
        x = jnp.concatenate(
            [x0[...], x1[...], x2[...], x3[...]], axis=0)
        eye = jnp.eye(128, dtype=jnp.float32)
        o_ref[...] = jax.lax.dot_general(
            x, eye, (((0,), (0,)), ((), ())),
            preferred_element_type=jnp.float32,
            precision=lax.Precision.HIGHEST)

    def in_spec(m):
        return pl.BlockSpec(
            (32, B4), lambda g, m=m: (0, jnp.minimum(g + m * NB, 488)))

    return pl.pallas_call(
        body,
        grid=(NB,),
        in_specs=[in_spec(m) for m in range(4)],
        out_specs=pl.BlockSpec((B4, 128), lambda g: (g, 0)),
        out_shape=jax.ShapeDtypeStruct((NB * B4, 128), jnp.float32),
        compiler_params=pltpu.CompilerParams(
            dimension_semantics=("arbitrary",)),
    )


def _make_gather(n_rows, dim, slots):
    mesh = plsc.VectorSubcoreMesh(core_axis_name="c", subcore_axis_name="s")

    @functools.partial(
        pl.kernel,
        mesh=mesh,
        out_type=jax.ShapeDtypeStruct((NW, slots, SLEN, dim), jnp.float32),
        scratch_types=[
            pltpu.VMEM((slots, SLEN), jnp.int32),
            pltpu.VMEM((NBUF, SLEN, dim), jnp.float32),
            [pltpu.SemaphoreType.DMA] * NBUF,
            [pltpu.SemaphoreType.DMA] * NBUF,
        ],
        compiler_params=pltpu.CompilerParams(use_tc_tiling_on_sc=False),
    )
    def grab(table_hbm, ids_hbm, out_hbm, idx_v, rows_v, gsems, osems):
        wid = lax.axis_index("s") * NC + lax.axis_index("c")
        pltpu.sync_copy(ids_hbm.at[wid], idx_v)

        def fire_gather(g):
            return pltpu.async_copy(
                table_hbm.at[idx_v.at[g]], rows_v.at[g % NBUF], gsems[g % NBUF]
            )

        gh = {g: fire_gather(g) for g in range(min(2, slots))}
        oh = {}
        for g in range(slots):
            gh.pop(g).wait()
            oh[g] = pltpu.async_copy(
                rows_v.at[g % NBUF], out_hbm.at[wid, g], osems[g % NBUF]
            )
            if g + 2 < slots:
                if g - 1 >= 0:
                    oh.pop(g - 1).wait()
                gh[g + 2] = fire_gather(g + 2)
        for h in oh.values():
            h.wait()

    return grab


def kernel(input, table):
    ids = input
    n = ids.shape[0] * ids.shape[1]
    dim = table.shape[1]
    slots = n // (NW * SLEN)
    flat = ids.reshape(-1)
    flat = 4 * (flat % 251904) + flat // 251904
    ids3 = flat.reshape(NW, slots, SLEN)
    tt = table.T
    packed = _tc_transpose()(tt, tt, tt, tt)
    tlin = packed.reshape(4 * 251904, dim)
    out = _make_gather(4 * 251904, dim, slots)(tlin, ids3)
    return out.reshape(ids.shape + (dim,))


# speedup vs baseline: 4.2662x; 1.0881x over previous
import functools

import jax
import jax.numpy as jnp
from jax import lax
from jax.experimental import pallas as pl
from jax.experimental.pallas import tpu as pltpu
from jax.experimental.pallas import tpu_sc as plsc

NC = 2
NS = 16
NW = NC * NS

VOCAB = 1000000
DIM = 32
GRP = 4

SLEN = 1024
NBUF = 3


def _tc_transpose():
    B4 = 4096
    NB = 62

    def body(x0, x1, x2, x3, o_ref):
        x = jnp.concatenate(
            [x0[...], x1[...], x2[...], x3[...]], axis=0)
        eye = jnp.eye(128, dtype=jnp.float32)
        o_ref[...] = jax.lax.dot_general(
            x, eye, (((0,), (0,)), ((), ())),
            preferred_element_type=jnp.float32,
            precision=lax.Precision.HIGHEST)

    def in_spec(m):
        return pl.BlockSpec(
            (32, B4), lambda g, m=m: (0, jnp.minimum(g + m * NB, 244)))

    return pl.pallas_call(
        body,
        grid=(NB,),
        in_specs=[in_spec(m) for m in range(4)],
        out_specs=pl.BlockSpec((B4, 128), lambda g: (g, 0)),
        out_shape=jax.ShapeDtypeStruct((NB * B4, 128), jnp.float32),
        compiler_params=pltpu.CompilerParams(
            dimension_semantics=("arbitrary",)),
    )


def _make_gather(n_rows, dim, slots):
    mesh = plsc.VectorSubcoreMesh(core_axis_name="c", subcore_axis_name="s")

    @functools.partial(
        pl.kernel,
        mesh=mesh,
        out_type=jax.ShapeDtypeStruct((NW, slots, SLEN, dim), jnp.float32),
        scratch_types=[
            pltpu.VMEM((slots, SLEN), jnp.int32),
            pltpu.VMEM((NBUF, SLEN, dim), jnp.float32),
            [pltpu.SemaphoreType.DMA] * NBUF,
            [pltpu.SemaphoreType.DMA] * NBUF,
        ],
        compiler_params=pltpu.CompilerParams(use_tc_tiling_on_sc=False),
    )
    def grab(table_hbm, ids_hbm, out_hbm, idx_v, rows_v, gsems, osems):
        wid = lax.axis_index("s") * NC + lax.axis_index("c")
        pltpu.sync_copy(ids_hbm.at[wid], idx_v)

        def fire_gather(g):
            return pltpu.async_copy(
                table_hbm.at[idx_v.at[g]], rows_v.at[g % NBUF], gsems[g % NBUF]
            )

        gh = {g: fire_gather(g) for g in range(min(2, slots))}
        oh = {}
        for g in range(slots):
            gh.pop(g).wait()
            oh[g] = pltpu.async_copy(
                rows_v.at[g % NBUF], out_hbm.at[wid, g], osems[g % NBUF]
            )
            if g + 2 < slots:
                if g - 1 >= 0:
                    oh.pop(g - 1).wait()
                gh[g + 2] = fire_gather(g + 2)
        for h in oh.values():
            h.wait()

    return grab


def kernel(input, table):
    ids = input
    n = ids.shape[0] * ids.shape[1]
    dim = table.shape[1]
    slots = n // (NW * SLEN)
    flat = ids.reshape(-1)
    flat = 4 * (flat % 253952) + flat // 253952
    ids3 = flat.reshape(NW, slots, SLEN)
    tt = table.T
    packed = _tc_transpose()(tt, tt, tt, tt)
    tlin = packed.reshape(4 * 253952, dim)
    out = _make_gather(4 * 253952, dim, slots)(tlin, ids3)
    return out.reshape(ids.shape + (dim,))


# speedup vs baseline: 4.4627x; 1.0461x over previous
import functools

import jax
import jax.numpy as jnp
from jax import lax
from jax.experimental import pallas as pl
from jax.experimental.pallas import tpu as pltpu
from jax.experimental.pallas import tpu_sc as plsc

NC = 2
NS = 16
NW = NC * NS

VOCAB = 1000000
DIM = 32
GRP = 4

SLEN = 1024
NBUF = 3


def _tc_transpose():
    B4 = 8192
    NB = 31

    def body(x0, x1, x2, x3, o_ref):
        x = jnp.concatenate(
            [x0[...], x1[...], x2[...], x3[...]], axis=0)
        eye = jnp.eye(128, dtype=jnp.float32)
        o_ref[...] = jax.lax.dot_general(
            x, eye, (((0,), (0,)), ((), ())),
            preferred_element_type=jnp.float32,
            precision=lax.Precision.HIGHEST)

    def in_spec(m):
        return pl.BlockSpec(
            (32, B4), lambda g, m=m: (0, jnp.minimum(g + m * NB, 122)))

    return pl.pallas_call(
        body,
        grid=(NB,),
        in_specs=[in_spec(m) for m in range(4)],
        out_specs=pl.BlockSpec((B4, 128), lambda g: (g, 0)),
        out_shape=jax.ShapeDtypeStruct((NB * B4, 128), jnp.float32),
        compiler_params=pltpu.CompilerParams(
            dimension_semantics=("arbitrary",)),
    )


def _make_gather(n_rows, dim, slots):
    mesh = plsc.VectorSubcoreMesh(core_axis_name="c", subcore_axis_name="s")

    @functools.partial(
        pl.kernel,
        mesh=mesh,
        out_type=jax.ShapeDtypeStruct((NW, slots, SLEN, dim), jnp.float32),
        scratch_types=[
            pltpu.VMEM((slots, SLEN), jnp.int32),
            pltpu.VMEM((NBUF, SLEN, dim), jnp.float32),
            [pltpu.SemaphoreType.DMA] * NBUF,
            [pltpu.SemaphoreType.DMA] * NBUF,
        ],
        compiler_params=pltpu.CompilerParams(use_tc_tiling_on_sc=False),
    )
    def grab(table_hbm, ids_hbm, out_hbm, idx_v, rows_v, gsems, osems):
        wid = lax.axis_index("s") * NC + lax.axis_index("c")
        pltpu.sync_copy(ids_hbm.at[wid], idx_v)

        def fire_gather(g):
            return pltpu.async_copy(
                table_hbm.at[idx_v.at[g]], rows_v.at[g % NBUF], gsems[g % NBUF]
            )

        gh = {g: fire_gather(g) for g in range(min(2, slots))}
        oh = {}
        for g in range(slots):
            gh.pop(g).wait()
            oh[g] = pltpu.async_copy(
                rows_v.at[g % NBUF], out_hbm.at[wid, g], osems[g % NBUF]
            )
            if g + 2 < slots:
                if g - 1 >= 0:
                    oh.pop(g - 1).wait()
                gh[g + 2] = fire_gather(g + 2)
        for h in oh.values():
            h.wait()

    return grab


def kernel(input, table):
    ids = input
    n = ids.shape[0] * ids.shape[1]
    dim = table.shape[1]
    slots = n // (NW * SLEN)
    flat = ids.reshape(-1)
    flat = 4 * (flat % 253952) + flat // 253952
    ids3 = flat.reshape(NW, slots, SLEN)
    tt = table.T
    packed = _tc_transpose()(tt, tt, tt, tt)
    tlin = packed.reshape(4 * 253952, dim)
    out = _make_gather(4 * 253952, dim, slots)(tlin, ids3)
    return out.reshape(ids.shape + (dim,))
